# baseline (device time: 31968 ns/iter reference)
import functools
import os

import jax
import jax.numpy as jnp
from jax import lax
from jax.experimental import pallas as pl
from jax.experimental.pallas import tpu as pltpu

_ABLATE = os.environ.get("ABLATE", "")

N_DEV = 8
N_TOK = 2048
D_IN = 512
H_OUT = 1024
E_TOTAL = 64
E_LOCAL = E_TOTAL // N_DEV
CAP = 25
SLOTS = 32
ROWS = E_LOCAL * SLOTS
TOK_PER = N_TOK // N_DEV
QMAX = 64
CROWS = N_DEV * QMAX

_BF = jnp.bfloat16
_F32 = jnp.float32


def _moe_body(ric_ref, rir_ref, x_hbm, w_hbm, out_ref,
              xv, wv, scol_ref, cbuf, sendbuf, crecv, recvbuf,
              x_dma_sem, w_dma_sem,
              c_send_sems, c_recv_sems, d_send_sems, d_recv_sems):
    my_pos = lax.axis_index("i")

    x_dma = pltpu.make_async_copy(x_hbm, xv, x_dma_sem)
    x_dma.start()
    w_dma = pltpu.make_async_copy(w_hbm, wv, w_dma_sem)
    w_dma.start()

    if _ABLATE != "nocomm":
        barrier_sem = pltpu.get_barrier_semaphore()
        for j in range(N_DEV):
            @pl.when(j != my_pos)
            def _(j=j):
                pl.semaphore_signal(
                    barrier_sem, inc=1,
                    device_id=(j,), device_id_type=pl.DeviceIdType.MESH,
                )
        pl.semaphore_wait(barrier_sem, N_DEV - 1)

    e_col = ric_ref[...]
    oh = (e_col == lax.broadcasted_iota(jnp.int32, (N_TOK, E_TOTAL), 1)).astype(
        _BF
    )
    L = (
        lax.broadcasted_iota(jnp.int32, (N_TOK, N_TOK), 0)
        >= lax.broadcasted_iota(jnp.int32, (N_TOK, N_TOK), 1)
    ).astype(_BF)
    pos = jnp.dot(L, oh, preferred_element_type=_F32)
    s_col = (
        jnp.sum(pos * oh.astype(_F32), axis=1, keepdims=True) - 1.0
    ).astype(jnp.int32)
    scol_ref[...] = s_col

    e_row = rir_ref[...]
    ohT = (e_row == lax.broadcasted_iota(jnp.int32, (E_TOTAL, N_TOK), 0)).astype(
        _BF
    )
    U = (
        lax.broadcasted_iota(jnp.int32, (N_TOK, N_TOK), 0)
        <= lax.broadcasted_iota(jnp.int32, (N_TOK, N_TOK), 1)
    ).astype(_BF)
    posT = jnp.dot(ohT, U, preferred_element_type=_F32)
    s_row = jnp.sum(posT * ohT.astype(_F32), axis=0, keepdims=True) - 1.0

    d_row = (e_row // E_LOCAL == my_pos) & (s_row < CAP)
    d_bf = d_row.astype(_BF)

    t0 = lax.broadcasted_iota(jnp.int32, (N_TOK, N_TOK), 0)
    t1 = lax.broadcasted_iota(jnp.int32, (N_TOK, N_TOK), 1)
    Ublk = ((t0 <= t1) & (t0 // TOK_PER == t1 // TOK_PER)).astype(_BF)
    q_row = jnp.dot(d_bf, Ublk, preferred_element_type=_F32) - 1.0

    Bm = (
        lax.broadcasted_iota(jnp.int32, (N_TOK, N_DEV), 0) // TOK_PER
        == lax.broadcasted_iota(jnp.int32, (N_TOK, N_DEV), 1)
    ).astype(_BF)
    Qs = jnp.dot(d_bf, Bm, preferred_element_type=_F32)
    max_q = jnp.max(Qs)

    r_iota = lax.broadcasted_iota(jnp.int32, (N_TOK, ROWS), 1)
    Gt = (
        (e_col == my_pos * E_LOCAL + r_iota // SLOTS)
        & (s_col == r_iota % SLOTS)
        & (r_iota % SLOTS < CAP)
    ).astype(_BF)

    x_dma.wait()
    xb = xv[...].astype(_BF)
    xg = lax.dot_general(
        Gt, xb, (((0,), (0,)), ((), ())),
        preferred_element_type=_F32,
    ).astype(_BF)

    w_dma.wait()
    ys = []
    for k in range(E_LOCAL):
        a = xg[k * SLOTS:(k + 1) * SLOTS, :]
        w = wv[k].astype(_BF)
        ys.append(jnp.dot(a, w, preferred_element_type=_F32))
    ybuf = jnp.concatenate(ys, axis=0).astype(_BF)

    u_iota = lax.broadcasted_iota(jnp.int32, (CROWS, N_TOK), 0)
    tu_iota = lax.broadcasted_iota(jnp.int32, (CROWS, N_TOK), 1)
    PK = (
        (tu_iota // TOK_PER == u_iota // QMAX)
        & (q_row == (u_iota % QMAX).astype(_F32))
        & d_row
    ).astype(_BF)
    PKG = jnp.dot(PK, Gt, preferred_element_type=_F32).astype(_BF)
    cbuf[...] = jnp.dot(PKG, ybuf, preferred_element_type=_F32).astype(_BF)

    @pl.when(max_q > float(QMAX))
    def _():
        sendbuf[...] = jnp.dot(
            Gt, ybuf, preferred_element_type=_F32
        ).astype(_BF)

    if _ABLATE == "nocomm":
        out_ref[...] = _unpack(my_pos, my_pos, ric_ref, scol_ref,
                               cbuf, sendbuf, local=True)
        return

    lane8 = lax.broadcasted_iota(jnp.int32, (1, N_DEV), 1)
    for o in range(1, N_DEV):
        j = (my_pos + o) % N_DEV
        q_j = jnp.sum(jnp.where(lane8 == j, Qs, 0.0))
        ov = q_j > float(QMAX)

        @pl.when(jnp.logical_not(ov))
        def _(j=j):
            rdma = pltpu.make_async_remote_copy(
                src_ref=cbuf.at[pl.ds(j * QMAX, QMAX)],
                dst_ref=crecv.at[pl.ds(my_pos * QMAX, QMAX)],
                send_sem=c_send_sems.at[j],
                recv_sem=c_recv_sems.at[my_pos],
                device_id=(j,),
                device_id_type=pl.DeviceIdType.MESH,
            )
            rdma.start()

        @pl.when(ov)
        def _(j=j):
            rdma = pltpu.make_async_remote_copy(
                src_ref=sendbuf.at[pl.ds(j * TOK_PER, TOK_PER)],
                dst_ref=recvbuf.at[pl.ds(my_pos * TOK_PER, TOK_PER)],
                send_sem=d_send_sems.at[j],
                recv_sem=d_recv_sems.at[my_pos],
                device_id=(j,),
                device_id_type=pl.DeviceIdType.MESH,
            )
            rdma.start()

    total = _unpack(my_pos, my_pos, ric_ref, scol_ref, cbuf, sendbuf,
                    local=True)

    for o in range(1, N_DEV):
        j = (my_pos + N_DEV - o) % N_DEV
        flag = (ric_ref[pl.ds(my_pos * TOK_PER, TOK_PER), :] // E_LOCAL == j) & (
            scol_ref[pl.ds(my_pos * TOK_PER, TOK_PER), :] < CAP
        )
        q_in = jnp.sum(flag.astype(_F32))
        ov = q_in > float(QMAX)

        @pl.when(jnp.logical_not(ov))
        def _(j=j):
            rdma = pltpu.make_async_remote_copy(
                src_ref=cbuf.at[pl.ds(0, QMAX)],
                dst_ref=crecv.at[pl.ds(j * QMAX, QMAX)],
                send_sem=c_send_sems.at[0],
                recv_sem=c_recv_sems.at[j],
                device_id=(j,),
                device_id_type=pl.DeviceIdType.MESH,
            )
            rdma.wait_recv()

        @pl.when(ov)
        def _(j=j):
            rdma = pltpu.make_async_remote_copy(
                src_ref=cbuf.at[pl.ds(0, QMAX)],
                dst_ref=recvbuf.at[pl.ds(j * TOK_PER, TOK_PER)],
                send_sem=c_send_sems.at[0],
                recv_sem=d_recv_sems.at[j],
                device_id=(j,),
                device_id_type=pl.DeviceIdType.MESH,
            )
            rdma.wait_recv()

        contrib = _unpack(my_pos, j, ric_ref, scol_ref, crecv, recvbuf,
                          local=False, flag=flag, ov=ov)
        total = total + contrib
    out_ref[...] = total

    for o in range(1, N_DEV):
        j = (my_pos + o) % N_DEV
        q_j = jnp.sum(jnp.where(lane8 == j, Qs, 0.0))
        ov = q_j > float(QMAX)

        @pl.when(jnp.logical_not(ov))
        def _(j=j):
            rdma = pltpu.make_async_remote_copy(
                src_ref=cbuf.at[pl.ds(j * QMAX, QMAX)],
                dst_ref=crecv.at[pl.ds(0, QMAX)],
                send_sem=c_send_sems.at[j],
                recv_sem=c_recv_sems.at[0],
                device_id=(j,),
                device_id_type=pl.DeviceIdType.MESH,
            )
            rdma.wait_send()

        @pl.when(ov)
        def _(j=j):
            rdma = pltpu.make_async_remote_copy(
                src_ref=sendbuf.at[pl.ds(j * TOK_PER, TOK_PER)],
                dst_ref=recvbuf.at[pl.ds(0, TOK_PER)],
                send_sem=d_send_sems.at[j],
                recv_sem=d_recv_sems.at[0],
                device_id=(j,),
                device_id_type=pl.DeviceIdType.MESH,
            )
            rdma.wait_send()

    @functools.partial(pl.run_scoped, second_barrier=pltpu.SemaphoreType.REGULAR)
    def _(second_barrier):
        for j in range(N_DEV):
            @pl.when(j != my_pos)
            def _(j=j):
                pl.semaphore_signal(
                    second_barrier, inc=1,
                    device_id=(j,), device_id_type=pl.DeviceIdType.MESH,
                )
        pl.semaphore_wait(second_barrier, N_DEV - 1)


def _unpack(my_pos, j, ric_ref, scol_ref, compact, dense, local,
            flag=None, ov=None):
    if flag is None:
        flag = (ric_ref[pl.ds(my_pos * TOK_PER, TOK_PER), :] // E_LOCAL == j) & (
            scol_ref[pl.ds(my_pos * TOK_PER, TOK_PER), :] < CAP
        )
    f_bf = flag.astype(_BF)
    L256 = (
        lax.broadcasted_iota(jnp.int32, (TOK_PER, TOK_PER), 0)
        >= lax.broadcasted_iota(jnp.int32, (TOK_PER, TOK_PER), 1)
    ).astype(_BF)
    q_mine = jnp.dot(L256, f_bf, preferred_element_type=_F32) - 1.0
    if ov is None:
        ov = jnp.sum(f_bf.astype(_F32)) > float(QMAX)
    S = (
        flag & (q_mine == lax.broadcasted_iota(
            jnp.int32, (TOK_PER, QMAX), 1).astype(_F32))
    ).astype(_BF)
    slot = my_pos if local else j
    c_val = jnp.dot(
        S, compact[pl.ds(slot * QMAX, QMAX), :],
        preferred_element_type=_F32,
    )
    d_val = dense[pl.ds(slot * TOK_PER, TOK_PER), :].astype(_F32)
    return jnp.where(ov, d_val, c_val)


def kernel(x, router_W, route_idx, expert_W):
    ri = route_idx.astype(jnp.int32)
    return pl.pallas_call(
        _moe_body,
        out_shape=jax.ShapeDtypeStruct((TOK_PER, H_OUT), jnp.float32),
        in_specs=[
            pl.BlockSpec(memory_space=pltpu.VMEM),
            pl.BlockSpec(memory_space=pltpu.VMEM),
            pl.BlockSpec(memory_space=pl.ANY),
            pl.BlockSpec(memory_space=pl.ANY),
        ],
        out_specs=pl.BlockSpec(memory_space=pltpu.VMEM),
        scratch_shapes=[
            pltpu.VMEM((N_TOK, D_IN), jnp.float32),
            pltpu.VMEM((E_LOCAL, D_IN, H_OUT), jnp.float32),
            pltpu.VMEM((N_TOK, 1), jnp.int32),
            pltpu.VMEM((CROWS, H_OUT), _BF),
            pltpu.VMEM((N_TOK, H_OUT), _BF),
            pltpu.VMEM((CROWS, H_OUT), _BF),
            pltpu.VMEM((N_TOK, H_OUT), _BF),
            pltpu.SemaphoreType.DMA,
            pltpu.SemaphoreType.DMA,
            pltpu.SemaphoreType.DMA((N_DEV,)),
            pltpu.SemaphoreType.DMA((N_DEV,)),
            pltpu.SemaphoreType.DMA((N_DEV,)),
            pltpu.SemaphoreType.DMA((N_DEV,)),
        ],
        compiler_params=pltpu.CompilerParams(
            collective_id=None if _ABLATE == "nocomm" else 0,
            vmem_limit_bytes=100 * 1024 * 1024,
        ),
    )(ri, ri.reshape(1, N_TOK), x, expert_W)
